# baseline (speedup 1.0000x reference)
import functools

import jax
import jax.numpy as jnp
from jax import lax
from jax.experimental import pallas as pl
from jax.experimental.pallas import tpu as pltpu
from jax.experimental.pallas import tpu_sc as plsc

BATCH = 1024
SEQ = 200
D = 64
VOCAB = 1000000
NC, NS, L = 2, 16, 16
NW = NC * NS
N_ITEMS = (SEQ // 8) * (BATCH // 128)
SCALE = 8.0

_mesh = plsc.VectorSubcoreMesh(
    core_axis_name="c", subcore_axis_name="s", num_cores=NC, num_subcores=NS
)


@functools.partial(
    pl.kernel,
    out_type=jax.ShapeDtypeStruct((SEQ, 8, 8, 8, 128), jnp.float32),
    mesh=_mesh,
    scratch_types=[
        pltpu.VMEM((8, 128), jnp.int32),
        pltpu.VMEM((128, D), jnp.float32),
        pltpu.VMEM((128, D), jnp.float32),
        pltpu.VMEM((8, 8, 128), jnp.float32),
        pltpu.VMEM((8, 8, 128), jnp.float32),
        pltpu.SemaphoreType.DMA,
        pltpu.SemaphoreType.DMA,
        pltpu.SemaphoreType.DMA,
        pltpu.SemaphoreType.DMA,
    ],
    compiler_params=pltpu.CompilerParams(
        use_tc_tiling_on_sc=False, needs_layout_passes=False
    ),
)
def _embed(
    x4_hbm, table_hbm, out_hbm,
    idx_v, rows0, rows1, ob0, ob1, sem0, sem1, osem0, osem1,
):
    wid = lax.axis_index("s") * NC + lax.axis_index("c")
    nitems = jnp.where(wid < N_ITEMS % NW, N_ITEMS // NW + 1, N_ITEMS // NW)
    rows = (rows0, rows1)
    obs = (ob0, ob1)
    sems = (sem0, sem1)
    osems = (osem0, osem1)
    lane = lax.iota(jnp.int32, L)
    rowvecs = [lane + bg * L for bg in range(128 // L)]

    @pl.loop(0, nitems)
    def _item(j):
        item = wid + NW * j
        st = item // 8
        bt = item % 8
        pltpu.sync_copy(x4_hbm.at[st, bt], idx_v)
        copies = [None, None]
        ocopies = [None, None]
        copies[0] = pltpu.async_copy(table_hbm.at[idx_v.at[0]], rows0, sem0)
        for ssub in range(8):
            p = ssub % 2
            copies[p].wait()
            if ssub < 7:
                q = (ssub + 1) % 2
                copies[q] = pltpu.async_copy(
                    table_hbm.at[idx_v.at[ssub + 1]], rows[q], sems[q]
                )
            rbuf = rows[p]
            ob = obs[p]
            if ocopies[p] is not None:
                ocopies[p].wait()

            @pl.loop(0, 8)
            def _dt(dt):
                for dsub in range(8):
                    col = jnp.full((L,), dt * 8 + dsub, jnp.int32)
                    for bg in range(128 // L):
                        v = plsc.load_gather(rbuf, [rowvecs[bg], col])
                        ob[dt, dsub, pl.ds(bg * L, L)] = v * SCALE

            ocopies[p] = pltpu.async_copy(
                ob, out_hbm.at[st * 8 + ssub, :, bt], osems[p]
            )
        ocopies[0].wait()
        ocopies[1].wait()


def kernel(x, table):
    x4 = x.T.reshape(SEQ // 8, 8, BATCH // 128, 128).transpose(0, 2, 1, 3)
    out5 = _embed(x4, table)
    return out5.transpose(2, 4, 0, 1, 3).reshape(BATCH, SEQ, D)


# speedup vs baseline: 1.3300x; 1.3300x over previous
import functools

import jax
import jax.numpy as jnp
from jax import lax
from jax.experimental import pallas as pl
from jax.experimental.pallas import tpu as pltpu
from jax.experimental.pallas import tpu_sc as plsc

BATCH = 1024
SEQ = 200
D = 64
VOCAB = 1000000
NC, NS, L = 2, 16, 16
NW = NC * NS
N_ITEMS = (SEQ // 8) * (BATCH // 128)
SCALE = 8.0

_mesh = plsc.VectorSubcoreMesh(
    core_axis_name="c", subcore_axis_name="s", num_cores=NC, num_subcores=NS
)


@functools.partial(
    pl.kernel,
    out_type=jax.ShapeDtypeStruct((SEQ, 8, 8, 8, 128), jnp.float32),
    mesh=_mesh,
    scratch_types=[
        pltpu.VMEM((8, 128), jnp.int32),
        pltpu.VMEM((128, D), jnp.float32),
        pltpu.VMEM((128, D), jnp.float32),
        pltpu.VMEM((8, 8, 132), jnp.float32),
        pltpu.VMEM((8, 8, 132), jnp.float32),
        pltpu.SemaphoreType.DMA,
        pltpu.SemaphoreType.DMA,
        pltpu.SemaphoreType.DMA,
        pltpu.SemaphoreType.DMA,
    ],
    compiler_params=pltpu.CompilerParams(
        use_tc_tiling_on_sc=False, needs_layout_passes=False
    ),
)
def _embed(
    x4_hbm, table_hbm, out_hbm,
    idx_v, rows0, rows1, ob0, ob1, sem0, sem1, osem0, osem1,
):
    wid = lax.axis_index("s") * NC + lax.axis_index("c")
    nitems = jnp.where(wid < N_ITEMS % NW, N_ITEMS // NW + 1, N_ITEMS // NW)
    rows = (rows0, rows1)
    obs = (ob0, ob1)
    sems = (sem0, sem1)
    osems = (osem0, osem1)
    lane = lax.iota(jnp.int32, L)
    dts = [(lane + dg * L) // 8 for dg in range(D // L)]
    dsubs = [(lane + dg * L) % 8 for dg in range(D // L)]

    @pl.loop(0, nitems)
    def _item(j):
        item = wid + NW * j
        st = item // 8
        bt = item % 8
        pltpu.sync_copy(x4_hbm.at[st, bt], idx_v)
        copies = [None, None]
        ocopies = [None, None]
        copies[0] = pltpu.async_copy(table_hbm.at[idx_v.at[0]], rows0, sem0)
        for ssub in range(8):
            p = ssub % 2
            copies[p].wait()
            if ssub < 7:
                q = (ssub + 1) % 2
                copies[q] = pltpu.async_copy(
                    table_hbm.at[idx_v.at[ssub + 1]], rows[q], sems[q]
                )
            rbuf = rows[p]
            ob = obs[p]
            if ocopies[p] is not None:
                ocopies[p].wait()

            @pl.loop(0, 128)
            def _tok(b):
                bvec = jnp.full((L,), b, jnp.int32)
                for dg in range(D // L):
                    v = rbuf[b, pl.ds(dg * L, L)] * SCALE
                    plsc.store_scatter(ob, [dts[dg], dsubs[dg], bvec], v)

            ocopies[p] = pltpu.async_copy(
                ob.at[:, :, pl.ds(0, 128)],
                out_hbm.at[st * 8 + ssub, :, bt],
                osems[p],
            )
        ocopies[0].wait()
        ocopies[1].wait()


def kernel(x, table):
    x4 = x.T.reshape(SEQ // 8, 8, BATCH // 128, 128).transpose(0, 2, 1, 3)
    out5 = _embed(x4, table)
    return out5.transpose(2, 4, 0, 1, 3).reshape(BATCH, SEQ, D)


# speedup vs baseline: 1.3364x; 1.0048x over previous
import functools

import jax
import jax.numpy as jnp
from jax import lax
from jax.experimental import pallas as pl
from jax.experimental.pallas import tpu as pltpu
from jax.experimental.pallas import tpu_sc as plsc

BATCH = 1024
SEQ = 200
D = 64
VOCAB = 1000000
NC, NS, L = 2, 16, 16
NW = NC * NS
N_ITEMS = (SEQ // 8) * (BATCH // 128)
SCALE = 8.0

_mesh = plsc.VectorSubcoreMesh(
    core_axis_name="c", subcore_axis_name="s", num_cores=NC, num_subcores=NS
)


@functools.partial(
    pl.kernel,
    out_type=jax.ShapeDtypeStruct((SEQ, 8, 8, 8, 128), jnp.float32),
    mesh=_mesh,
    scratch_types=[
        pltpu.VMEM((8, 128), jnp.int32),
        pltpu.VMEM((128, D), jnp.float32),
        pltpu.VMEM((128, D), jnp.float32),
        pltpu.VMEM((8, 8, 132), jnp.float32),
        pltpu.VMEM((8, 8, 132), jnp.float32),
        pltpu.SemaphoreType.DMA,
        pltpu.SemaphoreType.DMA,
        pltpu.SemaphoreType.DMA,
        pltpu.SemaphoreType.DMA,
    ],
    compiler_params=pltpu.CompilerParams(
        use_tc_tiling_on_sc=False, needs_layout_passes=False
    ),
)
def _embed(
    x4_hbm, table_hbm, out_hbm,
    idx_v, rows0, rows1, ob0, ob1, sem0, sem1, osem0, osem1,
):
    wid = lax.axis_index("s") * NC + lax.axis_index("c")
    nitems = jnp.where(wid < N_ITEMS % NW, N_ITEMS // NW + 1, N_ITEMS // NW)
    rows = (rows0, rows1)
    obs = (ob0, ob1)
    sems = (sem0, sem1)
    osems = (osem0, osem1)
    lane = lax.iota(jnp.int32, L)
    dts = [(lane + dg * L) // 8 for dg in range(D // L)]
    dsubs = [(lane + dg * L) % 8 for dg in range(D // L)]

    @pl.loop(0, nitems)
    def _item(j):
        item = wid + NW * j
        st = item // 8
        bt = item % 8
        pltpu.sync_copy(x4_hbm.at[st, bt], idx_v)
        copies = [None, None]
        ocopies = [None, None]
        copies[0] = pltpu.async_copy(table_hbm.at[idx_v.at[0]], rows0, sem0)
        for ssub in range(8):
            p = ssub % 2
            copies[p].wait()
            if ssub < 7:
                q = (ssub + 1) % 2
                copies[q] = pltpu.async_copy(
                    table_hbm.at[idx_v.at[ssub + 1]], rows[q], sems[q]
                )
            rbuf = rows[p]
            ob = obs[p]
            if ocopies[p] is not None:
                ocopies[p].wait()

            @pl.loop(0, 32)
            def _tok(b4):
                b0 = b4 * 4
                bvec0 = jnp.full((L,), b0, jnp.int32)
                for u in range(4):
                    b = b0 + u
                    bvec = bvec0 + u
                    for dg in range(D // L):
                        v = rbuf[b, pl.ds(dg * L, L)] * SCALE
                        plsc.store_scatter(ob, [dts[dg], dsubs[dg], bvec], v)

            ocopies[p] = pltpu.async_copy(
                ob.at[:, :, pl.ds(0, 128)],
                out_hbm.at[st * 8 + ssub, :, bt],
                osems[p],
            )
        ocopies[0].wait()
        ocopies[1].wait()


def kernel(x, table):
    x4 = x.T.reshape(SEQ // 8, 8, BATCH // 128, 128).transpose(0, 2, 1, 3)
    out5 = _embed(x4, table)
    return out5.transpose(2, 4, 0, 1, 3).reshape(BATCH, SEQ, D)
